# baseline (speedup 1.0000x reference)
import functools

import jax
import jax.numpy as jnp
from jax import lax
from jax.experimental import pallas as pl
from jax.experimental.pallas import tpu as pltpu
from jax.experimental.pallas import tpu_sc as plsc

NUM_CORES = 2
NUM_SUBCORES = 16
NUM_WORKERS = NUM_CORES * NUM_SUBCORES
LANES = 16
CB = 2
NBUF = 2


def _embed_sc(xf, tab_flat, batch, seq, dim):
    mesh = plsc.VectorSubcoreMesh(core_axis_name="c", subcore_axis_name="s")
    vd = tab_flat.shape[0]
    bpw = batch // NUM_WORKERS
    n_chunks = bpw // CB
    n_blocks = n_chunks // NBUF
    rpc = CB * seq
    groups = rpc // LANES

    @functools.partial(
        pl.kernel,
        out_type=jax.ShapeDtypeStruct((batch, seq, dim), jnp.float32),
        mesh=mesh,
        scratch_types=[
            pltpu.VMEM((vd,), jnp.float32),
            pltpu.VMEM((rpc,), jnp.int32),
            pltpu.VMEM((rpc,), jnp.int32),
            pltpu.VMEM((CB, seq, dim), jnp.float32),
            pltpu.VMEM((CB, seq, dim), jnp.float32),
            pltpu.SemaphoreType.DMA,
            pltpu.SemaphoreType.DMA,
            pltpu.SemaphoreType.DMA,
            pltpu.SemaphoreType.DMA,
        ],
        compiler_params=pltpu.CompilerParams(needs_layout_passes=False),
    )
    def k(xf_hbm, tab_hbm, out_hbm, tab_v, idx0, idx1, out0, out1, si0, si1, so0, so1):
        idx_b = (idx0, idx1)
        out_b = (out0, out1)
        sem_i = (si0, si1)
        sem_o = (so0, so1)
        wid = lax.axis_index("s") * NUM_CORES + lax.axis_index("c")
        nbase = wid * (bpw * seq)
        bbase = wid * bpw
        pltpu.sync_copy(tab_hbm, tab_v)

        for b in range(NBUF):
            pltpu.async_copy(
                xf_hbm.at[pl.ds(nbase + b * rpc, rpc)], idx_b[b], sem_i[b]
            )

        def blk_body(blk, carry):
            for b in range(NBUF):
                i = blk * NBUF + b
                boff = bbase + i * CB
                pltpu.make_async_copy(
                    xf_hbm.at[pl.ds(nbase + i * rpc, rpc)], idx_b[b], sem_i[b]
                ).wait()

                @pl.when(blk > 0)
                def _wait_out():
                    pltpu.make_async_copy(
                        out_b[b], out_hbm.at[pl.ds(0, CB), :, :], sem_o[b]
                    ).wait()

                def grp_body(g, c):
                    toffv = idx_b[b][pl.ds(g * LANES, LANES)] * dim
                    for j in range(LANES):
                        toff = toffv[j]
                        rr = g * LANES + j
                        bi = rr // seq
                        ss = rr % seq
                        for h in range(dim // LANES):
                            out_b[b][bi, ss, pl.ds(h * LANES, LANES)] = tab_v[
                                pl.ds(toff + h * LANES, LANES)
                            ]
                    return c

                lax.fori_loop(0, groups, grp_body, 0)
                pltpu.async_copy(
                    out_b[b], out_hbm.at[pl.ds(boff, CB), :, :], sem_o[b]
                )

                @pl.when(blk < n_blocks - 1)
                def _prefetch():
                    pltpu.async_copy(
                        xf_hbm.at[pl.ds(nbase + (i + NBUF) * rpc, rpc)],
                        idx_b[b],
                        sem_i[b],
                    )

            return carry

        lax.fori_loop(0, n_blocks, blk_body, 0)
        for b in range(NBUF):
            pltpu.make_async_copy(
                out_b[b], out_hbm.at[pl.ds(0, CB), :, :], sem_o[b]
            ).wait()

    return k(xf, tab_flat)


def kernel(x, table):
    batch, seq = x.shape
    vocab, dim = table.shape
    assert batch % (NUM_WORKERS * CB * NBUF) == 0
    xf = x.reshape(batch * seq).astype(jnp.int32)
    return _embed_sc(xf, table.reshape(vocab * dim), batch, seq, dim)


# speedup vs baseline: 1.3247x; 1.3247x over previous
import functools

import jax
import jax.numpy as jnp
from jax import lax
from jax.experimental import pallas as pl
from jax.experimental.pallas import tpu as pltpu
from jax.experimental.pallas import tpu_sc as plsc

NUM_CORES = 2
NUM_SUBCORES = 16
NUM_WORKERS = NUM_CORES * NUM_SUBCORES
LANES = 16
CB = 2
NBUF = 2


def _embed_sc(xf, tab_flat, batch, seq, dim):
    mesh = plsc.VectorSubcoreMesh(core_axis_name="c", subcore_axis_name="s")
    vd = tab_flat.shape[0]
    bpw = batch // NUM_WORKERS
    n_chunks = bpw // CB
    n_blocks = n_chunks // NBUF
    rpc = CB * seq
    groups = rpc // LANES

    @functools.partial(
        pl.kernel,
        out_type=jax.ShapeDtypeStruct((batch, seq, dim), jnp.float32),
        mesh=mesh,
        scratch_types=[
            pltpu.VMEM((vd,), jnp.float32),
            pltpu.VMEM((rpc,), jnp.int32),
            pltpu.VMEM((rpc,), jnp.int32),
            pltpu.VMEM((CB, seq, dim), jnp.float32),
            pltpu.VMEM((CB, seq, dim), jnp.float32),
            pltpu.SemaphoreType.DMA,
            pltpu.SemaphoreType.DMA,
            pltpu.SemaphoreType.DMA,
            pltpu.SemaphoreType.DMA,
        ],
        compiler_params=pltpu.CompilerParams(needs_layout_passes=False),
    )
    def k(xf_hbm, tab_hbm, out_hbm, tab_v, idx0, idx1, out0, out1, si0, si1, so0, so1):
        idx_b = (idx0, idx1)
        out_b = (out0, out1)
        sem_i = (si0, si1)
        sem_o = (so0, so1)
        wid = lax.axis_index("s") * NUM_CORES + lax.axis_index("c")
        nbase = wid * (bpw * seq)
        bbase = wid * bpw
        pltpu.sync_copy(tab_hbm, tab_v)

        for b in range(NBUF):
            pltpu.async_copy(
                xf_hbm.at[pl.ds(nbase + b * rpc, rpc)], idx_b[b], sem_i[b]
            )

        def blk_body(blk, carry):
            for b in range(NBUF):
                i = blk * NBUF + b
                boff = bbase + i * CB
                pltpu.make_async_copy(
                    xf_hbm.at[pl.ds(nbase + i * rpc, rpc)], idx_b[b], sem_i[b]
                ).wait()

                @pl.when(blk > 0)
                def _wait_out():
                    pltpu.make_async_copy(
                        out_b[b], out_hbm.at[pl.ds(0, CB), :, :], sem_o[b]
                    ).wait()

                def grp_body(g, c):
                    toffv = idx_b[b][pl.ds(g * LANES, LANES)] * dim
                    for j in range(LANES):
                        toff = toffv[j]
                        rr = g * LANES + j
                        bi = rr // seq
                        ss = rr % seq
                        for h in range(dim // LANES):
                            out_b[b][bi, ss, pl.ds(h * LANES, LANES)] = tab_v[
                                pl.ds(toff + h * LANES, LANES)
                            ]
                    return c

                lax.fori_loop(0, groups, grp_body, 0)
                pltpu.async_copy(
                    out_b[b], out_hbm.at[pl.ds(boff, CB), :, :], sem_o[b]
                )

                @pl.when(blk < n_blocks - 1)
                def _prefetch():
                    pltpu.async_copy(
                        xf_hbm.at[pl.ds(nbase + (i + NBUF) * rpc, rpc)],
                        idx_b[b],
                        sem_i[b],
                    )

            return carry

        lax.fori_loop(0, n_blocks, blk_body, 0)
        for b in range(NBUF):
            pltpu.make_async_copy(
                out_b[b], out_hbm.at[pl.ds(0, CB), :, :], sem_o[b]
            ).wait()

    return k(xf, tab_flat)


def kernel(x, table):
    batch, seq = x.shape
    vocab, dim = table.shape
    assert batch % (NUM_WORKERS * CB * NBUF) == 0
    xf = x.reshape(batch * seq).astype(jnp.int32)
    out = _embed_sc(xf, table.reshape(vocab * dim), batch, seq, dim)
    return lax.optimization_barrier(out)
